# baseline (speedup 1.0000x reference)
import functools

import jax
import jax.numpy as jnp
from jax import lax
from jax.experimental import pallas as pl
from jax.experimental.pallas import tpu as pltpu
from jax.experimental.pallas import tpu_sc as plsc

_NN = 100000
_NE = 6400000
_NC, _NS, _L = 2, 16, 16
_NW = _NC * _NS
_NP = 100352
_SLICE = _NP // _NS
_CH = 2000
_NCH = 100
_EW = _CH * _NCH
_W = 4096
_FP = 2048

_mesh = plsc.VectorSubcoreMesh(core_axis_name="c", subcore_axis_name="s")


@functools.partial(
    pl.kernel,
    out_type=jax.ShapeDtypeStruct((_NC, _NP), jnp.float32),
    mesh=_mesh,
    compiler_params=pltpu.CompilerParams(needs_layout_passes=False),
    scratch_types=[
        pltpu.VMEM((128,), jnp.float32),
        pltpu.VMEM((8, _L), jnp.float32),
        pltpu.VMEM((_NP,), jnp.float32),
        pltpu.VMEM((_CH,), jnp.float32),
        pltpu.VMEM((_CH,), jnp.float32),
        pltpu.VMEM((_CH,), jnp.int32),
        pltpu.VMEM((_CH,), jnp.int32),
        pltpu.VMEM((_CH,), jnp.int32),
        pltpu.VMEM((_CH,), jnp.int32),
        pltpu.VMEM((_W,), jnp.float32),
        pltpu.VMEM((_FP,), jnp.int32),
        pltpu.VMEM((_CH,), jnp.float32),
        pltpu.VMEM_SHARED((_NP,), jnp.float32),
        pltpu.SemaphoreType.DMA,
        pltpu.SemaphoreType.DMA,
    ],
)
def _zbl_sc(tab_hbm, cons_hbm, zqf_hbm, di_hbm, ii_hbm, ij_hbm, zeros_hbm,
            part_hbm, tab_v, cons_v, zpn_v, di0, di1, ii0, ii1, ij0, ij1,
            win_v, fidx_v, vfb_v, acc_sh, sem0, sem1):
    cid = lax.axis_index("c")
    sid = lax.axis_index("s")
    wid = sid * _NC + cid
    dis = (di0, di1)
    iis = (ii0, ii1)
    ijs = (ij0, ij1)
    sin = (sem0, sem1)

    def _fire(m, b):
        base = wid * _EW + m * _CH
        pltpu.async_copy(di_hbm.at[pl.ds(base, _CH)], dis[b], sin[b])
        pltpu.async_copy(ii_hbm.at[pl.ds(base, _CH)], iis[b], sin[b])
        pltpu.async_copy(ij_hbm.at[pl.ds(base, _CH)], ijs[b], sin[b])

    def _wait_in(b):
        pltpu.make_async_copy(di_hbm.at[pl.ds(0, _CH)], dis[b], sin[b]).wait()
        pltpu.make_async_copy(ii_hbm.at[pl.ds(0, _CH)], iis[b], sin[b]).wait()
        pltpu.make_async_copy(ij_hbm.at[pl.ds(0, _CH)], ijs[b], sin[b]).wait()

    _fire(0, 0)
    _fire(1, 1)

    pltpu.sync_copy(tab_hbm, tab_v)
    pltpu.sync_copy(cons_hbm, cons_v)
    pltpu.sync_copy(zeros_hbm.at[pl.ds(sid * _SLICE, _SLICE)],
                    acc_sh.at[pl.ds(sid * _SLICE, _SLICE)])
    pltpu.sync_copy(zeros_hbm.at[pl.ds(0, _W)], win_v)

    pltpu.sync_copy(zqf_hbm, zpn_v)

    @plsc.parallel_loop(0, _NP, step=_L, unroll=4)
    def _zbuild(i):
        z = plsc.bitcast(zpn_v[pl.ds(i, _L)], jnp.int32)
        zpn_v[pl.ds(i, _L)] = plsc.load_gather(tab_v, [z])

    plsc.subcore_barrier()

    c0 = cons_v[0]
    c1 = cons_v[1]
    c2 = cons_v[2]
    c3 = cons_v[3]
    na0 = cons_v[4]
    na1 = cons_v[5]
    na2 = cons_v[6]
    na3 = cons_v[7]
    iota = jnp.arange(_L, dtype=jnp.int32)
    idxp1 = jnp.minimum(iota + 1, _L - 1)
    lane15 = iota == (_L - 1)
    nlane15 = iota < (_L - 1)
    zero16 = jnp.zeros((_L,), jnp.float32)

    def _flush(wbase):
        def piece(q, _):
            pbase = wbase + q * _FP

            def bld(i, _):
                fidx_v[pl.ds(i * _L, _L)] = jnp.minimum(
                    iota + (pbase + i * _L), _NP - 1)
                return 0

            lax.fori_loop(0, _FP // _L, bld, 0, unroll=4)
            pltpu.sync_copy(win_v.at[pl.ds(q * _FP, _FP)],
                            acc_sh.at[fidx_v], add=True)

            def zro(i, _):
                win_v[pl.ds(q * _FP + i * _L, _L)] = zero16
                return 0

            lax.fori_loop(0, _FP // _L, zro, 0, unroll=4)
            return 0

        lax.fori_loop(0, _W // _FP, piece, 0)

    def _process(b, base):
        _wait_in(b)
        dib, iib, ijb = dis[b], iis[b], ijs[b]
        f = jnp.min(iib[pl.ds(0, _L)])
        l = jnp.max(iib[pl.ds(_CH - _L, _L)])
        need = l >= base + _W
        pl.when(need)(lambda: _flush(base))
        base = jnp.where(need, f, base)
        fits = l < base + _W
        basev = jnp.broadcast_to(base, (_L,))

        def fast():
            def evec(i, _):
                sl = pl.ds(i * _L, _L)
                k = iib[sl]
                si = plsc.load_gather(zpn_v, [k])
                sj = plsc.load_gather(zpn_v, [ijb[sl]])
                t = dib[sl] * (si + sj)
                v = c0 * jnp.exp(na0 * t)
                v = v + c1 * jnp.exp(na1 * t)
                v = v + c2 * jnp.exp(na2 * t)
                v = v + c3 * jnp.exp(na3 * t)
                s = plsc.cumsum(v)
                ks = jnp.take_along_axis(k, idxp1, axis=0)
                m_int = k != ks
                plsc.addupdate_scatter(win_v, [k - basev], s,
                                       mask=m_int | lane15)
                plsc.addupdate_scatter(win_v, [ks - basev], -s,
                                       mask=m_int & nlane15)
                return 0

            lax.fori_loop(0, _CH // _L, evec, 0, unroll=4)

        def slow():
            def evec(i, _):
                sl = pl.ds(i * _L, _L)
                si = plsc.load_gather(zpn_v, [iib[sl]])
                sj = plsc.load_gather(zpn_v, [ijb[sl]])
                t = dib[sl] * (si + sj)
                v = c0 * jnp.exp(na0 * t)
                v = v + c1 * jnp.exp(na1 * t)
                v = v + c2 * jnp.exp(na2 * t)
                v = v + c3 * jnp.exp(na3 * t)
                vfb_v[sl] = v
                return 0

            lax.fori_loop(0, _CH // _L, evec, 0, unroll=4)
            pltpu.sync_copy(vfb_v, acc_sh.at[iib], add=True)

        pl.when(fits)(fast)
        pl.when(jnp.logical_not(fits))(slow)
        return base

    def _pairs(g, base):
        for b in range(2):
            m = 2 * g + b
            base = _process(b, base)
            _fire(m + 2, b)
        return base

    base = lax.fori_loop(0, _NCH // 2 - 1, _pairs, jnp.int32(0))
    base = _process(0, base)
    base = _process(1, base)
    _flush(base)

    plsc.subcore_barrier()
    pltpu.sync_copy(acc_sh.at[pl.ds(sid * _SLICE, _SLICE)],
                    part_hbm.at[cid, pl.ds(sid * _SLICE, _SLICE)])


def _combine_body(p_ref, o_ref):
    o_ref[...] = p_ref[0] + p_ref[1]


_combine = pl.pallas_call(
    _combine_body,
    out_shape=jax.ShapeDtypeStruct((_NP,), jnp.float32),
)


def kernel(Z, Dij, idx_i, idx_j, p, d, c, a):
    f32 = jnp.float32
    zf = jnp.arange(128, dtype=f32)
    tab = (zf ** p).astype(f32) / d
    cons = jnp.concatenate(
        [jnp.broadcast_to(c.astype(f32)[:, None], (4, _L)),
         jnp.broadcast_to(-a.astype(f32)[:, None], (4, _L))], axis=0)
    zq = jnp.zeros((_NP,), jnp.int32).at[:_NN].set(Z.astype(jnp.int32))
    zqf = lax.bitcast_convert_type(zq, f32)
    di = Dij.astype(f32)
    ii = idx_i.astype(jnp.int32)
    ij = idx_j.astype(jnp.int32)
    zeros = jnp.zeros((_NP,), f32)
    part = _zbl_sc(tab, cons, zqf, di, ii, ij, zeros)
    return _combine(part)[:_NN]


# speedup vs baseline: 3.4398x; 3.4398x over previous
import functools

import jax
import jax.numpy as jnp
from jax import lax
from jax.experimental import pallas as pl
from jax.experimental.pallas import tpu as pltpu
from jax.experimental.pallas import tpu_sc as plsc

_NN = 100000
_NE = 6400000
_NC, _NS, _L = 2, 16, 16
_NW = _NC * _NS
_NP = 100352
_SLICE = _NP // _NS
_CH = 2000
_NCH = 100
_EW = _CH * _NCH
_W = 4096
_FP = 2048

_mesh = plsc.VectorSubcoreMesh(core_axis_name="c", subcore_axis_name="s")


@functools.partial(
    pl.kernel,
    out_type=jax.ShapeDtypeStruct((_NC, _NP), jnp.float32),
    mesh=_mesh,
    compiler_params=pltpu.CompilerParams(needs_layout_passes=False),
    scratch_types=[
        pltpu.VMEM((128,), jnp.float32),
        pltpu.VMEM((8, _L), jnp.float32),
        pltpu.VMEM((_NP,), jnp.float32),
        pltpu.VMEM((_CH,), jnp.float32),
        pltpu.VMEM((_CH,), jnp.float32),
        pltpu.VMEM((_CH,), jnp.int32),
        pltpu.VMEM((_CH,), jnp.int32),
        pltpu.VMEM((_CH,), jnp.int32),
        pltpu.VMEM((_CH,), jnp.int32),
        pltpu.VMEM((_W,), jnp.float32),
        pltpu.VMEM((_FP,), jnp.int32),
        pltpu.VMEM((_CH,), jnp.float32),
        pltpu.VMEM_SHARED((_NP,), jnp.float32),
        pltpu.SemaphoreType.DMA,
        pltpu.SemaphoreType.DMA,
    ],
)
def _zbl_sc(tab_hbm, cons_hbm, zqf_hbm, di_hbm, ii_hbm, ij_hbm, zeros_hbm,
            part_hbm, tab_v, cons_v, zpn_v, di0, di1, ii0, ii1, ij0, ij1,
            win_v, fidx_v, vfb_v, acc_sh, sem0, sem1):
    cid = lax.axis_index("c")
    sid = lax.axis_index("s")
    wid = sid * _NC + cid
    dis = (di0, di1)
    iis = (ii0, ii1)
    ijs = (ij0, ij1)
    sin = (sem0, sem1)

    def _fire(m, b):
        base = wid * _EW + m * _CH
        pltpu.async_copy(di_hbm.at[pl.ds(base, _CH)], dis[b], sin[b])
        pltpu.async_copy(ii_hbm.at[pl.ds(base, _CH)], iis[b], sin[b])
        pltpu.async_copy(ij_hbm.at[pl.ds(base, _CH)], ijs[b], sin[b])

    def _wait_in(b):
        pltpu.make_async_copy(di_hbm.at[pl.ds(0, _CH)], dis[b], sin[b]).wait()
        pltpu.make_async_copy(ii_hbm.at[pl.ds(0, _CH)], iis[b], sin[b]).wait()
        pltpu.make_async_copy(ij_hbm.at[pl.ds(0, _CH)], ijs[b], sin[b]).wait()

    _fire(0, 0)
    _fire(1, 1)

    pltpu.sync_copy(tab_hbm, tab_v)
    pltpu.sync_copy(cons_hbm, cons_v)
    pltpu.sync_copy(zeros_hbm.at[pl.ds(sid * _SLICE, _SLICE)],
                    acc_sh.at[pl.ds(sid * _SLICE, _SLICE)])
    pltpu.sync_copy(zeros_hbm.at[pl.ds(0, _W)], win_v)

    pltpu.sync_copy(zqf_hbm, zpn_v)

    @plsc.parallel_loop(0, _NP, step=_L, unroll=4)
    def _zbuild(i):
        z = plsc.bitcast(zpn_v[pl.ds(i, _L)], jnp.int32)
        zpn_v[pl.ds(i, _L)] = plsc.load_gather(tab_v, [z])

    plsc.subcore_barrier()

    c0 = cons_v[0]
    c1 = cons_v[1]
    c2 = cons_v[2]
    c3 = cons_v[3]
    na0 = cons_v[4]
    na1 = cons_v[5]
    na2 = cons_v[6]
    na3 = cons_v[7]
    iota = jnp.arange(_L, dtype=jnp.int32)
    idxp1 = jnp.minimum(iota + 1, _L - 1)
    lane15 = iota == (_L - 1)
    nlane15 = iota < (_L - 1)
    zero16 = jnp.zeros((_L,), jnp.float32)

    def _flush(wbase):
        def piece(q, _):
            pbase = wbase + q * _FP

            def bld(i, _):
                fidx_v[pl.ds(i * _L, _L)] = jnp.minimum(
                    iota + (pbase + i * _L), _NP - 1)
                return 0

            lax.fori_loop(0, _FP // _L, bld, 0, unroll=4)
            pltpu.sync_copy(win_v.at[pl.ds(q * _FP, _FP)],
                            acc_sh.at[fidx_v], add=True)

            def zro(i, _):
                win_v[pl.ds(q * _FP + i * _L, _L)] = zero16
                return 0

            lax.fori_loop(0, _FP // _L, zro, 0, unroll=4)
            return 0

        lax.fori_loop(0, _W // _FP, piece, 0)

    def _process(b, base):
        _wait_in(b)
        dib, iib, ijb = dis[b], iis[b], ijs[b]
        f = jnp.min(iib[pl.ds(0, _L)])
        l = jnp.max(iib[pl.ds(_CH - _L, _L)])
        need = l >= base + _W
        pl.when(need)(lambda: _flush(base))
        base = jnp.where(need, f, base)
        fits = l < base + _W
        basev = jnp.broadcast_to(base, (_L,))

        def fast():
            @plsc.parallel_loop(0, _CH, step=_L, unroll=4)
            def evec(i):
                sl = pl.ds(i, _L)
                k = iib[sl]
                si = plsc.load_gather(zpn_v, [k])
                sj = plsc.load_gather(zpn_v, [ijb[sl]])
                t = dib[sl] * (si + sj)
                v = c0 * jnp.exp(na0 * t)
                v = v + c1 * jnp.exp(na1 * t)
                v = v + c2 * jnp.exp(na2 * t)
                v = v + c3 * jnp.exp(na3 * t)
                s = plsc.cumsum(v)
                ks = jnp.take_along_axis(k, idxp1, axis=0)
                m_int = k != ks
                plsc.addupdate_scatter(win_v, [k - basev], s,
                                       mask=m_int | lane15)
                plsc.addupdate_scatter(win_v, [ks - basev], -s,
                                       mask=m_int & nlane15)

        def slow():
            def evec(i, _):
                sl = pl.ds(i * _L, _L)
                si = plsc.load_gather(zpn_v, [iib[sl]])
                sj = plsc.load_gather(zpn_v, [ijb[sl]])
                t = dib[sl] * (si + sj)
                v = c0 * jnp.exp(na0 * t)
                v = v + c1 * jnp.exp(na1 * t)
                v = v + c2 * jnp.exp(na2 * t)
                v = v + c3 * jnp.exp(na3 * t)
                vfb_v[sl] = v
                return 0

            lax.fori_loop(0, _CH // _L, evec, 0, unroll=4)
            pltpu.sync_copy(vfb_v, acc_sh.at[iib], add=True)

        pl.when(fits)(fast)
        pl.when(jnp.logical_not(fits))(slow)
        return base

    def _pairs(g, base):
        for b in range(2):
            m = 2 * g + b
            base = _process(b, base)
            _fire(m + 2, b)
        return base

    base = lax.fori_loop(0, _NCH // 2 - 1, _pairs, jnp.int32(0))
    base = _process(0, base)
    base = _process(1, base)
    _flush(base)

    plsc.subcore_barrier()
    pltpu.sync_copy(acc_sh.at[pl.ds(sid * _SLICE, _SLICE)],
                    part_hbm.at[cid, pl.ds(sid * _SLICE, _SLICE)])


def _combine_body(p_ref, o_ref):
    o_ref[...] = p_ref[0] + p_ref[1]


_combine = pl.pallas_call(
    _combine_body,
    out_shape=jax.ShapeDtypeStruct((_NP,), jnp.float32),
)


def kernel(Z, Dij, idx_i, idx_j, p, d, c, a):
    f32 = jnp.float32
    zf = jnp.arange(128, dtype=f32)
    tab = (zf ** p).astype(f32) / d
    cons = jnp.concatenate(
        [jnp.broadcast_to(c.astype(f32)[:, None], (4, _L)),
         jnp.broadcast_to(-a.astype(f32)[:, None], (4, _L))], axis=0)
    zq = jnp.zeros((_NP,), jnp.int32).at[:_NN].set(Z.astype(jnp.int32))
    zqf = lax.bitcast_convert_type(zq, f32)
    di = Dij.astype(f32)
    ii = idx_i.astype(jnp.int32)
    ij = idx_j.astype(jnp.int32)
    zeros = jnp.zeros((_NP,), f32)
    part = _zbl_sc(tab, cons, zqf, di, ii, ij, zeros)
    return _combine(part)[:_NN]
